# baseline (speedup 1.0000x reference)
import functools

import numpy as np
import jax
import jax.numpy as jnp
from jax import lax
from jax.experimental import pallas as pl
from jax.experimental.pallas import tpu as pltpu
from jax.experimental.pallas import tpu_sc as plsc

B = 16
N = 2048
S = 512
K = 32
RADIUS_SQ = 0.2 ** 2
R2F = np.float32(RADIUS_SQ)

F_IN = 64
D0 = 64
D1 = 64
D2 = 128


def _fps_body(px_ref, py_ref, pz_ref, ox_ref, oy_ref, oz_ref):
    px = px_ref[...]
    py = py_ref[...]
    pz = pz_ref[...]
    lane_n = lax.broadcasted_iota(jnp.int32, (B, N), 1)
    lane16 = lane_n.astype(jnp.int16)
    lane_s = lax.broadcasted_iota(jnp.int32, (B, S), 1)
    pall = jnp.concatenate([px, py, pz], axis=0)
    ox_ref[...] = jnp.zeros((B, S), jnp.float32)
    oy_ref[...] = jnp.zeros((B, S), jnp.float32)
    oz_ref[...] = jnp.zeros((B, S), jnp.float32)

    def body(i, state):
        dist, cx, cy, cz = state
        ox_ref[...] = jnp.where(lane_s == i, cx, ox_ref[...])
        oy_ref[...] = jnp.where(lane_s == i, cy, oy_ref[...])
        oz_ref[...] = jnp.where(lane_s == i, cz, oz_ref[...])
        dx = px - cx
        dy = py - cy
        dz = pz - cz
        d = (dx * dx + dy * dy) + dz * dz
        dist = jnp.minimum(dist, d)
        rd, ri, rx, ry, rz = dist, lane_n, px, py, pz
        w = N
        while w > 128:
            w //= 2
            c = rd[:, :w] >= rd[:, w:2 * w]
            rd = jnp.where(c, rd[:, :w], rd[:, w:2 * w])
            ri = jnp.where(c, ri[:, :w], ri[:, w:2 * w])
            rx = jnp.where(c, rx[:, :w], rx[:, w:2 * w])
            ry = jnp.where(c, ry[:, :w], ry[:, w:2 * w])
            rz = jnp.where(c, rz[:, :w], rz[:, w:2 * w])
        m = jnp.max(rd, axis=1, keepdims=True)
        idx = jnp.min(jnp.where(rd == m, ri, N), axis=1, keepdims=True)
        sel = ri == idx
        cx = jnp.sum(jnp.where(sel, rx, 0.0), axis=1, keepdims=True)
        cy = jnp.sum(jnp.where(sel, ry, 0.0), axis=1, keepdims=True)
        cz = jnp.sum(jnp.where(sel, rz, 0.0), axis=1, keepdims=True)
        return dist, cx, cy, cz

    dist0 = jnp.full((B, N), 1e10, jnp.float32)
    lax.fori_loop(0, S, body,
                  (dist0, px[:, 0:1], py[:, 0:1], pz[:, 0:1]), unroll=8)


def _fps(px, py, pz):
    out = jax.ShapeDtypeStruct((B, S), jnp.float32)
    return pl.pallas_call(
        _fps_body,
        out_shape=[out, out, out],
        in_specs=[pl.BlockSpec((B, N), lambda: (0, 0))] * 3,
        out_specs=[pl.BlockSpec((B, S), lambda: (0, 0))] * 3,
    )(px, py, pz)


_S_T = 512
_S_SUB = 256


def _ballq_body(b0, cx_ref, cy_ref, cz_ref, px_ref, py_ref, pz_ref, out_ref):
    b = pl.program_id(0) + b0
    dx = cx_ref[0, 0] - px_ref[0]
    dy = cy_ref[0, 0] - py_ref[0]
    dz = cz_ref[0, 0] - pz_ref[0]
    d = (dx * dx + dy * dy) + dz * dz
    mf = (d <= R2F).astype(jnp.float32)
    nb = N // 128
    m2 = mf.reshape(_S_T * nb, 128)
    lti = (lax.broadcasted_iota(jnp.int32, (128, 128), 0)
           <= lax.broadcasted_iota(jnp.int32, (128, 128), 1)).astype(jnp.float32)
    r1 = jnp.dot(m2, lti, preferred_element_type=jnp.float32)
    t3 = r1[:, 127:128].reshape(_S_T, nb)
    ult = (lax.broadcasted_iota(jnp.int32, (nb, nb), 0)
           < lax.broadcasted_iota(jnp.int32, (nb, nb), 1)).astype(jnp.float32)
    texc = jnp.dot(t3, ult, preferred_element_type=jnp.float32)
    rank = (r1.reshape(_S_T, nb, 128) + texc[:, :, None]).reshape(_S_T, N)
    r16 = rank.astype(jnp.int16)
    cols = []
    for k in range(K):
        t = (r16 <= jnp.int16(k)).astype(jnp.int16)
        w = N
        while w > 128:
            w //= 2
            t = t[:, :w] + t[:, w:2 * w]
        cols.append(t)
    tbig = jnp.concatenate(cols, axis=1).astype(jnp.float32)
    bd = (lax.broadcasted_iota(jnp.int32, (K * 128, K), 0) // 128
          == lax.broadcasted_iota(jnp.int32, (K * 128, K), 1)
          ).astype(jnp.float32)
    posf = jax.lax.dot_general(bd, tbig, (((0,), (1,)), ((), ())),
                               preferred_element_type=jnp.float32)
    posi = posf.astype(jnp.int32)
    res = jnp.where(posi < N, posi, posi[0:1, :]) + b * N
    for j in range(_S_T // _S_SUB):
        out_ref[0, j] = res[:, j * _S_SUB:(j + 1) * _S_SUB]


def _ball_query(ox, oy, oz, px, py, pz, b0):
    bh = ox.shape[0]
    grid = (bh, S // _S_T)
    c_spec = pl.BlockSpec((1, 1, _S_T, 1), lambda b, j: (b, j, 0, 0))
    p_spec = pl.BlockSpec((1, 1, N), lambda b, j: (b, 0, 0))
    c_shape = (bh, S // _S_T, _S_T, 1)
    p_shape = (bh, 1, N)
    nsub = _S_T // _S_SUB
    return pl.pallas_call(
        functools.partial(_ballq_body, b0),
        grid=grid,
        out_shape=jax.ShapeDtypeStruct((bh, nsub, K, _S_SUB), jnp.int32),
        in_specs=[c_spec, c_spec, c_spec, p_spec, p_spec, p_spec],
        out_specs=pl.BlockSpec((1, nsub, K, _S_SUB), lambda b, j: (b, 0, 0, 0)),
    )(ox.reshape(c_shape), oy.reshape(c_shape), oz.reshape(c_shape),
      px.reshape(p_shape), py.reshape(p_shape), pz.reshape(p_shape))


_CHUNK = 256
_IDX_MINOR = 128
_DG = 128


def _sc_gather(u_hbm, gidx):
    rows_total = gidx.shape[0]
    info = plsc.get_sparse_core_info()
    nw = info.num_cores * info.num_subcores
    chunks_total = rows_total // _CHUNK
    cpw = chunks_total // nw
    sub = _CHUNK // _IDX_MINOR
    idx3 = gidx.reshape(chunks_total, sub, _IDX_MINOR)
    mesh = plsc.VectorSubcoreMesh(core_axis_name="c", subcore_axis_name="s")

    @functools.partial(
        pl.kernel,
        mesh=mesh,
        out_type=jax.ShapeDtypeStruct((rows_total, _DG), jnp.float32),
        scratch_types=[
            pltpu.VMEM((2, sub, _IDX_MINOR), jnp.int32),
            pltpu.VMEM((2, _CHUNK, _DG), jnp.float32),
            pltpu.SemaphoreType.DMA,
            pltpu.SemaphoreType.DMA,
        ],
    )
    def gather_kernel(u_ref, idx_ref, out_ref, idx_v, rows_v, gsem, wsem):
        wid = lax.axis_index("s") * info.num_cores + lax.axis_index("c")
        base = wid * cpw

        def fire(c, sl):
            return [
                pltpu.async_copy(
                    u_ref.at[idx_v.at[sl, j]],
                    rows_v.at[sl, pl.ds(j * _IDX_MINOR, _IDX_MINOR)],
                    gsem,
                )
                for j in range(sub)
            ]

        pltpu.sync_copy(idx_ref.at[base], idx_v.at[0])
        gathers = {0: fire(0, 0)}
        writes = {}
        for c in range(cpw):
            sl = c % 2
            nsl = (c + 1) % 2
            for cp in gathers.pop(c):
                cp.wait()
            if c + 1 < cpw:
                if c - 1 in writes:
                    writes.pop(c - 1).wait()
                pltpu.sync_copy(idx_ref.at[base + c + 1], idx_v.at[nsl])
                gathers[c + 1] = fire(c + 1, nsl)
            writes[c] = pltpu.async_copy(
                rows_v.at[sl],
                out_ref.at[pl.ds((base + c) * _CHUNK, _CHUNK)],
                wsem,
            )
        for c in sorted(writes):
            writes.pop(c).wait()

    return gather_kernel(u_hbm, idx3)


_S_T_E = 256


def _mlp_body(g_ref, cx_ref, cy_ref, cz_ref, w0_ref, b0_ref, w1_ref, b1_ref,
              w2_ref, b2_ref, out_ref):
    w0 = w0_ref[...]
    offw = (cx_ref[...] * w0[0:1, :] + cy_ref[...] * w0[1:2, :]
            + cz_ref[...] * w0[2:3, :])
    c0 = b0_ref[...] - offw
    h0 = jnp.dot(g_ref[0], w0, preferred_element_type=jnp.float32)
    h0 = jnp.maximum(
        (h0.reshape(K, _S_T_E, D0) + c0[None, :, :]).reshape(K * _S_T_E, D0),
        0.0)
    h1 = jnp.maximum(
        jnp.dot(h0, w1_ref[...], preferred_element_type=jnp.float32)
        + b1_ref[...], 0.0)
    h2 = jnp.maximum(
        jnp.dot(h1, w2_ref[...], preferred_element_type=jnp.float32)
        + b2_ref[...], 0.0)
    h3 = h2.reshape(K, _S_T_E, D2)
    m = h3[0]
    for k in range(1, K):
        m = jnp.maximum(m, h3[k])
    out_ref[...] = m


def _mlp_pool(g3, cxe, cye, cze, w0pad, b0, w1, b1, w2, b2):
    rows = cxe.shape[0]
    grid = (rows // _S_T_E,)
    return pl.pallas_call(
        _mlp_body,
        grid=grid,
        out_shape=jax.ShapeDtypeStruct((rows, D2), jnp.float32),
        in_specs=[
            pl.BlockSpec((1, K * _S_T_E, _DG), lambda i: (i, 0, 0)),
            pl.BlockSpec((_S_T_E, 1), lambda i: (i, 0)),
            pl.BlockSpec((_S_T_E, 1), lambda i: (i, 0)),
            pl.BlockSpec((_S_T_E, 1), lambda i: (i, 0)),
            pl.BlockSpec((_DG, D0), lambda i: (0, 0)),
            pl.BlockSpec((1, D0), lambda i: (0, 0)),
            pl.BlockSpec((D0, D1), lambda i: (0, 0)),
            pl.BlockSpec((1, D1), lambda i: (0, 0)),
            pl.BlockSpec((D1, D2), lambda i: (0, 0)),
            pl.BlockSpec((1, D2), lambda i: (0, 0)),
        ],
        out_specs=pl.BlockSpec((_S_T_E, D2), lambda i: (i, 0)),
    )(g3, cxe, cye, cze, w0pad, b0, w1, b1, w2, b2)


def kernel(pos, feat, W0, b0, W1, b1, W2, b2):
    px = pos[:, :, 0]
    py = pos[:, :, 1]
    pz = pos[:, :, 2]
    ox, oy, oz = _fps(px, py, pz)

    u = jnp.concatenate(
        [pos, feat, jnp.zeros((B, N, _DG - 3 - F_IN), jnp.float32)],
        axis=-1).reshape(B * N, _DG)
    w0pad = jnp.concatenate(
        [W0, jnp.zeros((_DG - 3 - F_IN, D0), jnp.float32)], axis=0)

    jt = S // _S_T_E
    nh = 2
    bh = B // nh
    feats = []
    for h in range(nh):
        sl = slice(h * bh, (h + 1) * bh)
        gidx = _ball_query(ox[sl], oy[sl], oz[sl],
                           px[sl], py[sl], pz[sl], h * bh)
        g = _sc_gather(u, gidx.reshape(-1))
        feats.append(_mlp_pool(
            g.reshape(bh * jt, K * _S_T_E, _DG),
            ox[sl].reshape(bh * S, 1), oy[sl].reshape(bh * S, 1),
            oz[sl].reshape(bh * S, 1),
            w0pad, b0.reshape(1, D0), W1, b1.reshape(1, D1), W2,
            b2.reshape(1, D2)))

    new_feat = jnp.concatenate(feats, axis=0)
    new_pos = jnp.stack([ox, oy, oz], axis=-1)
    return new_pos, new_feat.reshape(B, S, D2)


# speedup vs baseline: 1.0029x; 1.0029x over previous
import functools

import numpy as np
import jax
import jax.numpy as jnp
from jax import lax
from jax.experimental import pallas as pl
from jax.experimental.pallas import tpu as pltpu
from jax.experimental.pallas import tpu_sc as plsc

B = 16
N = 2048
S = 512
K = 32
RADIUS_SQ = 0.2 ** 2
R2F = np.float32(RADIUS_SQ)

F_IN = 64
D0 = 64
D1 = 64
D2 = 128


def _fps_body(px_ref, py_ref, pz_ref, ox_ref, oy_ref, oz_ref):
    px = px_ref[...]
    py = py_ref[...]
    pz = pz_ref[...]
    lane_n = lax.broadcasted_iota(jnp.int32, (B, N), 1)
    lane16 = lane_n.astype(jnp.int16)
    lane_s = lax.broadcasted_iota(jnp.int32, (B, S), 1)
    pall = jnp.concatenate([px, py, pz], axis=0)
    ox_ref[...] = jnp.zeros((B, S), jnp.float32)
    oy_ref[...] = jnp.zeros((B, S), jnp.float32)
    oz_ref[...] = jnp.zeros((B, S), jnp.float32)

    def body(i, state):
        dist, cx, cy, cz = state
        ox_ref[...] = jnp.where(lane_s == i, cx, ox_ref[...])
        oy_ref[...] = jnp.where(lane_s == i, cy, oy_ref[...])
        oz_ref[...] = jnp.where(lane_s == i, cz, oz_ref[...])
        dx = px - cx
        dy = py - cy
        dz = pz - cz
        d = (dx * dx + dy * dy) + dz * dz
        dist = jnp.minimum(dist, d)
        rd, ri, rx, ry, rz = dist, lane_n, px, py, pz
        w = N
        while w > 128:
            w //= 2
            c = rd[:, :w] >= rd[:, w:2 * w]
            rd = jnp.where(c, rd[:, :w], rd[:, w:2 * w])
            ri = jnp.where(c, ri[:, :w], ri[:, w:2 * w])
            rx = jnp.where(c, rx[:, :w], rx[:, w:2 * w])
            ry = jnp.where(c, ry[:, :w], ry[:, w:2 * w])
            rz = jnp.where(c, rz[:, :w], rz[:, w:2 * w])
        m = jnp.max(rd, axis=1, keepdims=True)
        idx = jnp.min(jnp.where(rd == m, ri, N), axis=1, keepdims=True)
        sel = ri == idx
        cx = jnp.sum(jnp.where(sel, rx, 0.0), axis=1, keepdims=True)
        cy = jnp.sum(jnp.where(sel, ry, 0.0), axis=1, keepdims=True)
        cz = jnp.sum(jnp.where(sel, rz, 0.0), axis=1, keepdims=True)
        return dist, cx, cy, cz

    dist0 = jnp.full((B, N), 1e10, jnp.float32)
    lax.fori_loop(0, S, body,
                  (dist0, px[:, 0:1], py[:, 0:1], pz[:, 0:1]), unroll=8)


def _fps(px, py, pz):
    out = jax.ShapeDtypeStruct((B, S), jnp.float32)
    return pl.pallas_call(
        _fps_body,
        out_shape=[out, out, out],
        in_specs=[pl.BlockSpec((B, N), lambda: (0, 0))] * 3,
        out_specs=[pl.BlockSpec((B, S), lambda: (0, 0))] * 3,
    )(px, py, pz)


_S_T = 512
_S_SUB = 256


def _ballq_body(b0, cx_ref, cy_ref, cz_ref, px_ref, py_ref, pz_ref, out_ref):
    b = pl.program_id(0) + b0
    dx = cx_ref[0, 0] - px_ref[0]
    dy = cy_ref[0, 0] - py_ref[0]
    dz = cz_ref[0, 0] - pz_ref[0]
    d = (dx * dx + dy * dy) + dz * dz
    mf = (d <= R2F).astype(jnp.float32)
    nb = N // 128
    m2 = mf.reshape(_S_T * nb, 128)
    lti = (lax.broadcasted_iota(jnp.int32, (128, 128), 0)
           <= lax.broadcasted_iota(jnp.int32, (128, 128), 1)).astype(jnp.float32)
    r1 = jnp.dot(m2, lti, preferred_element_type=jnp.float32)
    t3 = r1[:, 127:128].reshape(_S_T, nb)
    ult = (lax.broadcasted_iota(jnp.int32, (nb, nb), 0)
           < lax.broadcasted_iota(jnp.int32, (nb, nb), 1)).astype(jnp.float32)
    texc = jnp.dot(t3, ult, preferred_element_type=jnp.float32)
    rank = (r1.reshape(_S_T, nb, 128) + texc[:, :, None]).reshape(_S_T, N)
    r16 = rank.astype(jnp.int16)
    cols = []
    for k in range(K):
        t = (r16 <= jnp.int16(k)).astype(jnp.int16)
        w = N
        while w > 128:
            w //= 2
            t = t[:, :w] + t[:, w:2 * w]
        cols.append(t)
    tbig = jnp.concatenate(cols, axis=1).astype(jnp.float32)
    bd = (lax.broadcasted_iota(jnp.int32, (K * 128, K), 0) // 128
          == lax.broadcasted_iota(jnp.int32, (K * 128, K), 1)
          ).astype(jnp.float32)
    posf = jax.lax.dot_general(bd, tbig, (((0,), (1,)), ((), ())),
                               preferred_element_type=jnp.float32)
    posi = posf.astype(jnp.int32)
    res = jnp.where(posi < N, posi, posi[0:1, :]) + b * N
    for j in range(_S_T // _S_SUB):
        out_ref[0, j] = res[:, j * _S_SUB:(j + 1) * _S_SUB]


def _ball_query(ox, oy, oz, px, py, pz, b0):
    bh = ox.shape[0]
    grid = (bh, S // _S_T)
    c_spec = pl.BlockSpec((1, 1, _S_T, 1), lambda b, j: (b, j, 0, 0))
    p_spec = pl.BlockSpec((1, 1, N), lambda b, j: (b, 0, 0))
    c_shape = (bh, S // _S_T, _S_T, 1)
    p_shape = (bh, 1, N)
    nsub = _S_T // _S_SUB
    return pl.pallas_call(
        functools.partial(_ballq_body, b0),
        grid=grid,
        out_shape=jax.ShapeDtypeStruct((bh, nsub, K, _S_SUB), jnp.int32),
        in_specs=[c_spec, c_spec, c_spec, p_spec, p_spec, p_spec],
        out_specs=pl.BlockSpec((1, nsub, K, _S_SUB), lambda b, j: (b, 0, 0, 0)),
    )(ox.reshape(c_shape), oy.reshape(c_shape), oz.reshape(c_shape),
      px.reshape(p_shape), py.reshape(p_shape), pz.reshape(p_shape))


_CHUNK = 256
_IDX_MINOR = 128
_DG = 128


def _sc_gather(u_hbm, gidx):
    rows_total = gidx.shape[0]
    info = plsc.get_sparse_core_info()
    nw = info.num_cores * info.num_subcores
    chunks_total = rows_total // _CHUNK
    cpw = chunks_total // nw
    sub = _CHUNK // _IDX_MINOR
    idx3 = gidx.reshape(chunks_total, sub, _IDX_MINOR)
    mesh = plsc.VectorSubcoreMesh(core_axis_name="c", subcore_axis_name="s")

    @functools.partial(
        pl.kernel,
        mesh=mesh,
        out_type=jax.ShapeDtypeStruct((rows_total, _DG), jnp.float32),
        scratch_types=[
            pltpu.VMEM((2, sub, _IDX_MINOR), jnp.int32),
            pltpu.VMEM((2, _CHUNK, _DG), jnp.float32),
            pltpu.SemaphoreType.DMA,
            pltpu.SemaphoreType.DMA,
        ],
    )
    def gather_kernel(u_ref, idx_ref, out_ref, idx_v, rows_v, gsem, wsem):
        wid = lax.axis_index("s") * info.num_cores + lax.axis_index("c")
        base = wid * cpw

        def fire(c, sl):
            return [
                pltpu.async_copy(
                    u_ref.at[idx_v.at[sl, j]],
                    rows_v.at[sl, pl.ds(j * _IDX_MINOR, _IDX_MINOR)],
                    gsem,
                )
                for j in range(sub)
            ]

        pltpu.sync_copy(idx_ref.at[base], idx_v.at[0])
        gathers = {0: fire(0, 0)}
        writes = {}
        for c in range(cpw):
            sl = c % 2
            nsl = (c + 1) % 2
            for cp in gathers.pop(c):
                cp.wait()
            if c + 1 < cpw:
                if c - 1 in writes:
                    writes.pop(c - 1).wait()
                pltpu.sync_copy(idx_ref.at[base + c + 1], idx_v.at[nsl])
                gathers[c + 1] = fire(c + 1, nsl)
            writes[c] = pltpu.async_copy(
                rows_v.at[sl],
                out_ref.at[pl.ds((base + c) * _CHUNK, _CHUNK)],
                wsem,
            )
        for c in sorted(writes):
            writes.pop(c).wait()

    return gather_kernel(u_hbm, idx3)


_S_T_E = 256


def _mlp_body(g_ref, cx_ref, cy_ref, cz_ref, w0_ref, b0_ref, w1_ref, b1_ref,
              w2_ref, b2_ref, out_ref):
    lane = lax.broadcasted_iota(jnp.int32, (_S_T_E, _DG), 1)
    off = (jnp.where(lane == 0, cx_ref[...], 0.0)
           + jnp.where(lane == 1, cy_ref[...], 0.0)
           + jnp.where(lane == 2, cz_ref[...], 0.0))
    g3 = g_ref[0].reshape(K, _S_T_E, _DG)
    x = (g3 - off[None, :, :]).reshape(K * _S_T_E, _DG)
    h0 = jnp.maximum(
        jnp.dot(x, w0_ref[...], preferred_element_type=jnp.float32)
        + b0_ref[...], 0.0)
    h1 = jnp.maximum(
        jnp.dot(h0, w1_ref[...], preferred_element_type=jnp.float32)
        + b1_ref[...], 0.0)
    h2 = jnp.maximum(
        jnp.dot(h1, w2_ref[...], preferred_element_type=jnp.float32)
        + b2_ref[...], 0.0)
    h3 = h2.reshape(K, _S_T_E, D2)
    m = h3[0]
    for k in range(1, K):
        m = jnp.maximum(m, h3[k])
    out_ref[...] = m


def _mlp_pool(g3, cxe, cye, cze, w0pad, b0, w1, b1, w2, b2):
    rows = cxe.shape[0]
    grid = (rows // _S_T_E,)
    return pl.pallas_call(
        _mlp_body,
        grid=grid,
        out_shape=jax.ShapeDtypeStruct((rows, D2), jnp.float32),
        in_specs=[
            pl.BlockSpec((1, K * _S_T_E, _DG), lambda i: (i, 0, 0)),
            pl.BlockSpec((_S_T_E, 1), lambda i: (i, 0)),
            pl.BlockSpec((_S_T_E, 1), lambda i: (i, 0)),
            pl.BlockSpec((_S_T_E, 1), lambda i: (i, 0)),
            pl.BlockSpec((_DG, D0), lambda i: (0, 0)),
            pl.BlockSpec((1, D0), lambda i: (0, 0)),
            pl.BlockSpec((D0, D1), lambda i: (0, 0)),
            pl.BlockSpec((1, D1), lambda i: (0, 0)),
            pl.BlockSpec((D1, D2), lambda i: (0, 0)),
            pl.BlockSpec((1, D2), lambda i: (0, 0)),
        ],
        out_specs=pl.BlockSpec((_S_T_E, D2), lambda i: (i, 0)),
    )(g3, cxe, cye, cze, w0pad, b0, w1, b1, w2, b2)


def kernel(pos, feat, W0, b0, W1, b1, W2, b2):
    px = pos[:, :, 0]
    py = pos[:, :, 1]
    pz = pos[:, :, 2]
    ox, oy, oz = _fps(px, py, pz)

    u = jnp.concatenate(
        [pos, feat, jnp.zeros((B, N, _DG - 3 - F_IN), jnp.float32)],
        axis=-1).reshape(B * N, _DG)
    w0pad = jnp.concatenate(
        [W0, jnp.zeros((_DG - 3 - F_IN, D0), jnp.float32)], axis=0)

    jt = S // _S_T_E
    nh = 2
    bh = B // nh
    feats = []
    for h in range(nh):
        sl = slice(h * bh, (h + 1) * bh)
        gidx = _ball_query(ox[sl], oy[sl], oz[sl],
                           px[sl], py[sl], pz[sl], h * bh)
        g = _sc_gather(u, gidx.reshape(-1))
        feats.append(_mlp_pool(
            g.reshape(bh * jt, K * _S_T_E, _DG),
            ox[sl].reshape(bh * S, 1), oy[sl].reshape(bh * S, 1),
            oz[sl].reshape(bh * S, 1),
            w0pad, b0.reshape(1, D0), W1, b1.reshape(1, D1), W2,
            b2.reshape(1, D2)))

    new_feat = jnp.concatenate(feats, axis=0)
    new_pos = jnp.stack([ox, oy, oz], axis=-1)
    return new_pos, new_feat.reshape(B, S, D2)
